# baseline (speedup 1.0000x reference)
import functools

import numpy as np
import jax
import jax.numpy as jnp
from jax import lax
from jax.experimental import pallas as pl
from jax.experimental.pallas import tpu as pltpu
from jax.experimental.pallas import tpu_sc as plsc

_B = 16384
_L = 200
_EMB = 8

_NC = 2
_NS = 16
_NW = _NC * _NS
_RPW = _B // _NW
_C = 8
_CHI = _C * _L
_NCHUNK = _RPW // _C

_NV = 1000001
_CW = 3072
_NBLK = _NV // _CW
_TAILC = _NV - _NBLK * _CW
_FLAT = _NV * _EMB


@functools.partial(
    pl.kernel,
    out_type=[jax.ShapeDtypeStruct((_FLAT,), jnp.float32)] * 3,
    mesh=plsc.VectorSubcoreMesh(core_axis_name="c", subcore_axis_name="s"),
    compiler_params=pltpu.CompilerParams(needs_layout_passes=False),
    scratch_types=(
        [pltpu.VMEM((_EMB, _CW), jnp.float32) for _ in range(2)]
        + [pltpu.VMEM((_CW * _EMB,), jnp.float32) for _ in range(2)]
        + [pltpu.VMEM((_EMB, _TAILC), jnp.float32)]
        + [pltpu.VMEM((_EMB * 17,), jnp.float32)]
        + [pltpu.SemaphoreType.DMA for _ in range(4)]
    ),
)
def _sc_compact(t0_h, t1_h, t2_h, o0_h, o1_h, o2_h,
                b20, b21, b10, b11, bt, st, si0, si1, so0, so1):
    wid = lax.axis_index("s") * _NC + lax.axis_index("c")
    b2 = (b20, b21)
    b1 = (b10, b11)
    si = (si0, si1)
    so = (so0, so1)
    iota = lax.iota(jnp.int32, 16)
    div8 = iota >> 3
    mod8 = iota & 7
    n_own = (_NBLK - 1 - wid) // _NW + 1

    for tab_h, out_h in ((t0_h, o0_h), (t1_h, o1_h), (t2_h, o2_h)):
        def in_desc(i, b, tab_h=tab_h):
            c0 = (wid + i * _NW) * _CW
            return pltpu.make_async_copy(
                tab_h.at[:, pl.ds(c0, _CW)], b2[b], si[b])

        def out_desc(i, b, out_h=out_h):
            c0 = (wid + i * _NW) * _CW
            return pltpu.make_async_copy(
                b1[b], out_h.at[pl.ds(c0 * _EMB, _CW * _EMB)], so[b])

        def transpose(b):
            stidx = mod8 * 17 + div8

            def body(q, c):
                c0 = q * 16
                for d in range(8):
                    st[pl.ds(d * 17, 16)] = b2[b][d, pl.ds(c0, 16)]
                for p in range(8):
                    v = plsc.load_gather(st, [stidx + 2 * p])
                    b1[b][pl.ds(c0 * _EMB + p * 16, 16)] = v
                return c
            lax.fori_loop(0, _CW // 16, body, 0)

        in_desc(0, 0).start()

        def step(i, c):
            for b in range(2):
                @pl.when(2 * i + b < n_own)
                def _(b=b, i=i):
                    ii = 2 * i + b
                    in_desc(ii, b).wait()
                    @pl.when(ii + 1 < n_own)
                    def _():
                        in_desc(ii + 1, 1 - b).start()
                    @pl.when(ii >= 2)
                    def _():
                        out_desc(ii - 2, b).wait()
                    transpose(b)
                    out_desc(ii, b).start()
            return c

        lax.fori_loop(0, (n_own + 1) // 2, step, 0)
        for b in range(2):
            @pl.when(n_own > b)
            def _(b=b):
                i_b = ((n_own - 1 - b) // 2) * 2 + b
                out_desc(i_b, b).wait()

        @pl.when(wid == _NW - 1)
        def _(tab_h=tab_h, out_h=out_h):
            c0 = _NBLK * _CW
            pltpu.make_async_copy(
                tab_h.at[:, pl.ds(c0, _TAILC)], bt, si[0]).start()
            pltpu.make_async_copy(
                tab_h.at[:, pl.ds(c0, _TAILC)], bt, si[0]).wait()
            def body(p, c):
                col = jnp.minimum(2 * p + div8, _TAILC - 1)
                v = plsc.load_gather(bt, [mod8, col])
                b1[0][pl.ds(p * 16, 16)] = v
                return c
            lax.fori_loop(0, (_TAILC * _EMB + 15) // 16, body, 0)
            pltpu.sync_copy(b1[0].at[pl.ds(0, _TAILC * _EMB)],
                            out_h.at[pl.ds(c0 * _EMB, _TAILC * _EMB)])


@functools.partial(
    pl.kernel,
    out_type=jax.ShapeDtypeStruct((_B * 48,), jnp.float32),
    mesh=plsc.VectorSubcoreMesh(core_axis_name="c", subcore_axis_name="s"),
    compiler_params=pltpu.CompilerParams(
        needs_layout_passes=False, use_tc_tiling_on_sc=False),
    scratch_types=(
        [pltpu.VMEM((_CHI,), jnp.int32) for _ in range(6)]
        + [pltpu.VMEM((_CHI,), jnp.float32) for _ in range(2)]
        + [pltpu.VMEM((_CHI, _EMB), jnp.float32) for _ in range(6)]
        + [pltpu.VMEM((_C * 48,), jnp.float32)]
        + [pltpu.SemaphoreType.DMA for _ in range(4)]
    ),
)
def _sc_pool(shp_h, clr_h, clu_h, msk_h, ts_h, tc_h, tu_h, out_h,
             i00, i01, i02, i10, i11, i12, m0, m1,
             d00, d01, d02, d10, d11, d12,
             out_v, sem_i0, sem_i1, sem_g0, sem_g1):
    wid = lax.axis_index("s") * _NC + lax.axis_index("c")
    base = wid * _RPW

    idx_hbm = (shp_h, clr_h, clu_h)
    tab_hbm = (ts_h, tc_h, tu_h)
    idx_v = ((i00, i01, i02), (i10, i11, i12))
    msk_v = (m0, m1)
    dat_v = ((d00, d01, d02), (d10, d11, d12))
    sems_i = (sem_i0, sem_i1)
    sems_g = (sem_g0, sem_g1)

    def in_descs(k, b):
        off = (base + k * _C) * _L
        ds = [pltpu.make_async_copy(idx_hbm[t].at[pl.ds(off, _CHI)],
                                    idx_v[b][t], sems_i[b]) for t in range(3)]
        ds.append(pltpu.make_async_copy(msk_h.at[pl.ds(off, _CHI)],
                                        msk_v[b], sems_i[b]))
        return ds

    _SUB = 128
    _NFULL = _CHI // _SUB
    _TAIL = _CHI - _NFULL * _SUB

    def _g_desc(b, t, off, n):
        return pltpu.make_async_copy(
            tab_hbm[t].at[idx_v[b][t].at[pl.ds(off, n)]],
            dat_v[b][t].at[pl.ds(off, n), :], sems_g[b])

    def start_gathers(b):
        def fire(s, c):
            for t in range(3):
                _g_desc(b, t, s * _SUB, _SUB).start()
            return c
        lax.fori_loop(0, _NFULL, fire, 0)
        for t in range(3):
            _g_desc(b, t, _NFULL * _SUB, _TAIL).start()

    def wait_gathers(b):
        def w(s, c):
            for t in range(3):
                _g_desc(b, t, 0, _SUB).wait()
            return c
        lax.fori_loop(0, _NFULL, w, 0)
        for t in range(3):
            _g_desc(b, t, 0, _TAIL).wait()

    def start(descs):
        for d in descs:
            d.start()

    def wait(descs):
        for d in descs:
            d.wait()

    iota = lax.iota(jnp.int32, 16)
    div8 = iota >> 3
    mod8 = iota & 7
    zero = jnp.zeros((16,), jnp.float32)

    def compute(k, b):
        def row_body(r, _):
            roff = r * _L

            def j_body(jc, accs):
                a0, a1, a2 = accs
                base = roff + 8 * jc + div8
                for u in range(4):
                    ridx = base + 2 * u
                    m = plsc.load_gather(msk_v[b], [ridx])
                    d0 = plsc.load_gather(dat_v[b][0], [ridx, mod8])
                    d1 = plsc.load_gather(dat_v[b][1], [ridx, mod8])
                    d2 = plsc.load_gather(dat_v[b][2], [ridx, mod8])
                    a0 = a0 + d0 * m
                    a1 = a1 + d1 * m
                    a2 = a2 + d2 * m
                return (a0, a1, a2)

            a0, a1, a2 = lax.fori_loop(0, _L // 8, j_body, (zero, zero, zero))
            out_v[pl.ds(r * 48, 16)] = a0
            out_v[pl.ds(r * 48 + 16, 16)] = a1
            out_v[pl.ds(r * 48 + 32, 16)] = a2
            return 0

        lax.fori_loop(0, _C, row_body, 0)
        pltpu.sync_copy(out_v, out_h.at[pl.ds((base + k * _C) * 48, _C * 48)])

    start(in_descs(0, 0))
    wait(in_descs(0, 0))
    start_gathers(0)
    start(in_descs(1, 1))

    def chunk_pair(k2, carry):
        for par in range(2):
            k = 2 * k2 + par
            b, nb = par, 1 - par

            @pl.when(k + 1 < _NCHUNK)
            def _():
                wait(in_descs(k + 1, nb))
                start_gathers(nb)

            wait_gathers(b)
            compute(k, b)

            @pl.when(k + 2 < _NCHUNK)
            def _():
                start(in_descs(k + 2, b))
        return carry

    lax.fori_loop(0, _NCHUNK // 2, chunk_pair, 0)


_FOLD = np.zeros((48, 24), np.float32)
for _t in range(3):
    for _p in range(2):
        for _d in range(8):
            _FOLD[_t * 16 + _p * 8 + _d, _t * 8 + _d] = 1.0

_BM = 512


def _mlp_body(x_ref, m_ref, w1_ref, b1_ref, w2_ref, b2_ref, o_ref):
    x = x_ref[...]
    xw = jnp.dot(x, w1_ref[...], preferred_element_type=jnp.float32)
    msum = jnp.sum(m_ref[...], axis=1, keepdims=True)
    h = jnp.maximum(xw / msum + b1_ref[...], 0.0)
    o_ref[...] = jnp.dot(h, w2_ref[...], preferred_element_type=jnp.float32) + b2_ref[...]


def kernel(shp, clr, clust, mask, shape_table, color_table, cluster_table,
           W1, b1, W2, b2):
    f0, f1, f2 = _sc_compact(shape_table.T, color_table.T, cluster_table.T)
    x48 = _sc_pool(shp.reshape(-1), clr.reshape(-1), clust.reshape(-1),
                   mask.reshape(-1), f0.reshape(_NV, _EMB),
                   f1.reshape(_NV, _EMB), f2.reshape(_NV, _EMB))
    x48 = x48.reshape(_B, 48)
    w1f = jnp.asarray(_FOLD) @ W1
    ncls = W2.shape[1]
    return pl.pallas_call(
        _mlp_body,
        grid=(_B // _BM,),
        in_specs=[
            pl.BlockSpec((_BM, 48), lambda i: (i, 0)),
            pl.BlockSpec((_BM, _L), lambda i: (i, 0)),
            pl.BlockSpec((48, 64), lambda i: (0, 0)),
            pl.BlockSpec((1, 64), lambda i: (0, 0)),
            pl.BlockSpec((64, ncls), lambda i: (0, 0)),
            pl.BlockSpec((1, ncls), lambda i: (0, 0)),
        ],
        out_specs=pl.BlockSpec((_BM, ncls), lambda i: (i, 0)),
        out_shape=jax.ShapeDtypeStruct((_B, ncls), jnp.float32),
    )(x48, mask, w1f, b1.reshape(1, -1), W2, b2.reshape(1, -1))


# speedup vs baseline: 1.0151x; 1.0151x over previous
import functools

import numpy as np
import jax
import jax.numpy as jnp
from jax import lax
from jax.experimental import pallas as pl
from jax.experimental.pallas import tpu as pltpu
from jax.experimental.pallas import tpu_sc as plsc

_B = 16384
_L = 200
_EMB = 8

_NC = 2
_NS = 16
_NW = _NC * _NS
_RPW = _B // _NW
_C = 8
_CHI = _C * _L
_NCHUNK = _RPW // _C

_NV = 1000001
_CW = 2048
_NBLK = _NV // _CW
_TAILC = _NV - _NBLK * _CW
_FLAT = _NV * _EMB


@functools.partial(
    pl.kernel,
    out_type=[jax.ShapeDtypeStruct((_FLAT,), jnp.float32)] * 3,
    mesh=plsc.VectorSubcoreMesh(core_axis_name="c", subcore_axis_name="s"),
    compiler_params=pltpu.CompilerParams(needs_layout_passes=False),
    scratch_types=(
        [pltpu.VMEM((_EMB, _CW), jnp.float32) for _ in range(2)]
        + [pltpu.VMEM((_CW * _EMB,), jnp.float32) for _ in range(2)]
        + [pltpu.VMEM((_EMB, _TAILC), jnp.float32)]
        + [pltpu.VMEM((_EMB * 17,), jnp.float32)]
        + [pltpu.SemaphoreType.DMA for _ in range(4)]
    ),
)
def _sc_compact(t0_h, t1_h, t2_h, o0_h, o1_h, o2_h,
                b20, b21, b10, b11, bt, st, si0, si1, so0, so1):
    wid = lax.axis_index("s") * _NC + lax.axis_index("c")
    b2 = (b20, b21)
    b1 = (b10, b11)
    si = (si0, si1)
    so = (so0, so1)
    iota = lax.iota(jnp.int32, 16)
    div8 = iota >> 3
    mod8 = iota & 7
    n_own = (_NBLK - 1 - wid) // _NW + 1

    for tab_h, out_h in ((t0_h, o0_h), (t1_h, o1_h), (t2_h, o2_h)):
        def in_desc(i, b, tab_h=tab_h):
            c0 = (wid + i * _NW) * _CW
            return pltpu.make_async_copy(
                tab_h.at[:, pl.ds(c0, _CW)], b2[b], si[b])

        def out_desc(i, b, out_h=out_h):
            c0 = (wid + i * _NW) * _CW
            return pltpu.make_async_copy(
                b1[b], out_h.at[pl.ds(c0 * _EMB, _CW * _EMB)], so[b])

        def transpose(b):
            stidx = mod8 * 17 + div8

            def body(q, c):
                c0 = q * 16
                for d in range(8):
                    st[pl.ds(d * 17, 16)] = b2[b][d, pl.ds(c0, 16)]
                for p in range(8):
                    v = plsc.load_gather(st, [stidx + 2 * p])
                    b1[b][pl.ds(c0 * _EMB + p * 16, 16)] = v
                return c
            lax.fori_loop(0, _CW // 16, body, 0)

        in_desc(0, 0).start()

        def step(i, c):
            for b in range(2):
                @pl.when(2 * i + b < n_own)
                def _(b=b, i=i):
                    ii = 2 * i + b
                    in_desc(ii, b).wait()
                    @pl.when(ii + 1 < n_own)
                    def _():
                        in_desc(ii + 1, 1 - b).start()
                    @pl.when(ii >= 2)
                    def _():
                        out_desc(ii - 2, b).wait()
                    transpose(b)
                    out_desc(ii, b).start()
            return c

        lax.fori_loop(0, (n_own + 1) // 2, step, 0)
        for b in range(2):
            @pl.when(n_own > b)
            def _(b=b):
                i_b = ((n_own - 1 - b) // 2) * 2 + b
                out_desc(i_b, b).wait()

        @pl.when(wid == _NW - 1)
        def _(tab_h=tab_h, out_h=out_h):
            c0 = _NBLK * _CW
            pltpu.make_async_copy(
                tab_h.at[:, pl.ds(c0, _TAILC)], bt, si[0]).start()
            pltpu.make_async_copy(
                tab_h.at[:, pl.ds(c0, _TAILC)], bt, si[0]).wait()
            def body(p, c):
                col = jnp.minimum(2 * p + div8, _TAILC - 1)
                v = plsc.load_gather(bt, [mod8, col])
                b1[0][pl.ds(p * 16, 16)] = v
                return c
            lax.fori_loop(0, (_TAILC * _EMB + 15) // 16, body, 0)
            pltpu.sync_copy(b1[0].at[pl.ds(0, _TAILC * _EMB)],
                            out_h.at[pl.ds(c0 * _EMB, _TAILC * _EMB)])


@functools.partial(
    pl.kernel,
    out_type=jax.ShapeDtypeStruct((_B * 48,), jnp.float32),
    mesh=plsc.VectorSubcoreMesh(core_axis_name="c", subcore_axis_name="s"),
    compiler_params=pltpu.CompilerParams(
        needs_layout_passes=False, use_tc_tiling_on_sc=False),
    scratch_types=(
        [pltpu.VMEM((_CHI,), jnp.int32) for _ in range(6)]
        + [pltpu.VMEM((_CHI,), jnp.float32) for _ in range(2)]
        + [pltpu.VMEM((_CHI, _EMB), jnp.float32) for _ in range(6)]
        + [pltpu.VMEM((_C * 48,), jnp.float32)]
        + [pltpu.SemaphoreType.DMA for _ in range(4)]
    ),
)
def _sc_pool(shp_h, clr_h, clu_h, msk_h, ts_h, tc_h, tu_h, out_h,
             i00, i01, i02, i10, i11, i12, m0, m1,
             d00, d01, d02, d10, d11, d12,
             out_v, sem_i0, sem_i1, sem_g0, sem_g1):
    wid = lax.axis_index("s") * _NC + lax.axis_index("c")
    base = wid * _RPW

    idx_hbm = (shp_h, clr_h, clu_h)
    tab_hbm = (ts_h, tc_h, tu_h)
    idx_v = ((i00, i01, i02), (i10, i11, i12))
    msk_v = (m0, m1)
    dat_v = ((d00, d01, d02), (d10, d11, d12))
    sems_i = (sem_i0, sem_i1)
    sems_g = (sem_g0, sem_g1)

    def in_descs(k, b):
        off = (base + k * _C) * _L
        ds = [pltpu.make_async_copy(idx_hbm[t].at[pl.ds(off, _CHI)],
                                    idx_v[b][t], sems_i[b]) for t in range(3)]
        ds.append(pltpu.make_async_copy(msk_h.at[pl.ds(off, _CHI)],
                                        msk_v[b], sems_i[b]))
        return ds

    _SUB = 128
    _NFULL = _CHI // _SUB
    _TAIL = _CHI - _NFULL * _SUB

    def _g_desc(b, t, off, n):
        return pltpu.make_async_copy(
            tab_hbm[t].at[idx_v[b][t].at[pl.ds(off, n)]],
            dat_v[b][t].at[pl.ds(off, n), :], sems_g[b])

    def start_gathers(b):
        def fire(s, c):
            for t in range(3):
                _g_desc(b, t, s * _SUB, _SUB).start()
            return c
        lax.fori_loop(0, _NFULL, fire, 0)
        for t in range(3):
            _g_desc(b, t, _NFULL * _SUB, _TAIL).start()

    def wait_gathers(b):
        def w(s, c):
            for t in range(3):
                _g_desc(b, t, 0, _SUB).wait()
            return c
        lax.fori_loop(0, _NFULL, w, 0)
        for t in range(3):
            _g_desc(b, t, 0, _TAIL).wait()

    def start(descs):
        for d in descs:
            d.start()

    def wait(descs):
        for d in descs:
            d.wait()

    iota = lax.iota(jnp.int32, 16)
    div8 = iota >> 3
    mod8 = iota & 7
    zero = jnp.zeros((16,), jnp.float32)

    def compute(k, b):
        def row_body(r, _):
            roff = r * _L

            def j_body(jc, accs):
                a0, a1, a2 = accs
                base = roff + 8 * jc + div8
                for u in range(4):
                    ridx = base + 2 * u
                    m = plsc.load_gather(msk_v[b], [ridx])
                    d0 = plsc.load_gather(dat_v[b][0], [ridx, mod8])
                    d1 = plsc.load_gather(dat_v[b][1], [ridx, mod8])
                    d2 = plsc.load_gather(dat_v[b][2], [ridx, mod8])
                    a0 = a0 + d0 * m
                    a1 = a1 + d1 * m
                    a2 = a2 + d2 * m
                return (a0, a1, a2)

            a0, a1, a2 = lax.fori_loop(0, _L // 8, j_body, (zero, zero, zero))
            out_v[pl.ds(r * 48, 16)] = a0
            out_v[pl.ds(r * 48 + 16, 16)] = a1
            out_v[pl.ds(r * 48 + 32, 16)] = a2
            return 0

        lax.fori_loop(0, _C, row_body, 0)
        pltpu.sync_copy(out_v, out_h.at[pl.ds((base + k * _C) * 48, _C * 48)])

    start(in_descs(0, 0))
    wait(in_descs(0, 0))
    start_gathers(0)
    start(in_descs(1, 1))

    def chunk_pair(k2, carry):
        for par in range(2):
            k = 2 * k2 + par
            b, nb = par, 1 - par

            @pl.when(k + 1 < _NCHUNK)
            def _():
                wait(in_descs(k + 1, nb))
                start_gathers(nb)

            wait_gathers(b)
            compute(k, b)

            @pl.when(k + 2 < _NCHUNK)
            def _():
                start(in_descs(k + 2, b))
        return carry

    lax.fori_loop(0, _NCHUNK // 2, chunk_pair, 0)


_FOLD = np.zeros((48, 24), np.float32)
for _t in range(3):
    for _p in range(2):
        for _d in range(8):
            _FOLD[_t * 16 + _p * 8 + _d, _t * 8 + _d] = 1.0

_BM = 512


def _mlp_body(x_ref, m_ref, w1_ref, b1_ref, w2_ref, b2_ref, o_ref):
    x = x_ref[...]
    xw = jnp.dot(x, w1_ref[...], preferred_element_type=jnp.float32)
    msum = jnp.sum(m_ref[...], axis=1, keepdims=True)
    h = jnp.maximum(xw / msum + b1_ref[...], 0.0)
    o_ref[...] = jnp.dot(h, w2_ref[...], preferred_element_type=jnp.float32) + b2_ref[...]


def kernel(shp, clr, clust, mask, shape_table, color_table, cluster_table,
           W1, b1, W2, b2):
    f0, f1, f2 = _sc_compact(shape_table.T, color_table.T, cluster_table.T)
    x48 = _sc_pool(shp.reshape(-1), clr.reshape(-1), clust.reshape(-1),
                   mask.reshape(-1), f0.reshape(_NV, _EMB),
                   f1.reshape(_NV, _EMB), f2.reshape(_NV, _EMB))
    x48 = x48.reshape(_B, 48)
    w1f = jnp.asarray(_FOLD) @ W1
    ncls = W2.shape[1]
    return pl.pallas_call(
        _mlp_body,
        grid=(_B // _BM,),
        in_specs=[
            pl.BlockSpec((_BM, 48), lambda i: (i, 0)),
            pl.BlockSpec((_BM, _L), lambda i: (i, 0)),
            pl.BlockSpec((48, 64), lambda i: (0, 0)),
            pl.BlockSpec((1, 64), lambda i: (0, 0)),
            pl.BlockSpec((64, ncls), lambda i: (0, 0)),
            pl.BlockSpec((1, ncls), lambda i: (0, 0)),
        ],
        out_specs=pl.BlockSpec((_BM, ncls), lambda i: (i, 0)),
        out_shape=jax.ShapeDtypeStruct((_B, ncls), jnp.float32),
    )(x48, mask, w1f, b1.reshape(1, -1), W2, b2.reshape(1, -1))


# speedup vs baseline: 1.0741x; 1.0581x over previous
import functools

import numpy as np
import jax
import jax.numpy as jnp
from jax import lax
from jax.experimental import pallas as pl
from jax.experimental.pallas import tpu as pltpu
from jax.experimental.pallas import tpu_sc as plsc

_B = 16384
_L = 200
_EMB = 8

_NC = 2
_NS = 16
_NW = _NC * _NS
_RPW = _B // _NW
_C = 8
_CHI = _C * _L
_NCHUNK = _RPW // _C

_NV = 1000001
_CW = 2048
_NBLK = _NV // _CW
_TAILC = _NV - _NBLK * _CW
_FLAT = _NV * _EMB


@functools.partial(
    pl.kernel,
    out_type=[jax.ShapeDtypeStruct((_FLAT,), jnp.float32)] * 3,
    mesh=plsc.VectorSubcoreMesh(core_axis_name="c", subcore_axis_name="s"),
    compiler_params=pltpu.CompilerParams(needs_layout_passes=False),
    scratch_types=(
        [pltpu.VMEM((_EMB, _CW), jnp.float32) for _ in range(2)]
        + [pltpu.VMEM((_CW * _EMB,), jnp.float32) for _ in range(2)]
        + [pltpu.VMEM((_EMB, _TAILC), jnp.float32)]
        + [pltpu.VMEM((_EMB * 17,), jnp.float32)]
        + [pltpu.SemaphoreType.DMA for _ in range(4)]
    ),
)
def _sc_compact(t0_h, t1_h, t2_h, o0_h, o1_h, o2_h,
                b20, b21, b10, b11, bt, st, si0, si1, so0, so1):
    wid = lax.axis_index("s") * _NC + lax.axis_index("c")
    b2 = (b20, b21)
    b1 = (b10, b11)
    si = (si0, si1)
    so = (so0, so1)
    iota = lax.iota(jnp.int32, 16)
    div8 = iota >> 3
    mod8 = iota & 7
    n_own = (_NBLK - 1 - wid) // _NW + 1

    for tab_h, out_h in ((t0_h, o0_h), (t1_h, o1_h), (t2_h, o2_h)):
        def in_desc(i, b, tab_h=tab_h):
            c0 = (wid + i * _NW) * _CW
            return pltpu.make_async_copy(
                tab_h.at[:, pl.ds(c0, _CW)], b2[b], si[b])

        def out_desc(i, b, out_h=out_h):
            c0 = (wid + i * _NW) * _CW
            return pltpu.make_async_copy(
                b1[b], out_h.at[pl.ds(c0 * _EMB, _CW * _EMB)], so[b])

        def transpose(b):
            stidx = mod8 * 17 + div8

            def body(q, c):
                c0 = q * 16
                for d in range(8):
                    st[pl.ds(d * 17, 16)] = b2[b][d, pl.ds(c0, 16)]
                for p in range(8):
                    v = plsc.load_gather(st, [stidx + 2 * p])
                    b1[b][pl.ds(c0 * _EMB + p * 16, 16)] = v
                return c
            lax.fori_loop(0, _CW // 16, body, 0)

        in_desc(0, 0).start()

        def step(i, c):
            for b in range(2):
                @pl.when(2 * i + b < n_own)
                def _(b=b, i=i):
                    ii = 2 * i + b
                    in_desc(ii, b).wait()
                    @pl.when(ii + 1 < n_own)
                    def _():
                        in_desc(ii + 1, 1 - b).start()
                    @pl.when(ii >= 2)
                    def _():
                        out_desc(ii - 2, b).wait()
                    transpose(b)
                    out_desc(ii, b).start()
            return c

        lax.fori_loop(0, (n_own + 1) // 2, step, 0)
        for b in range(2):
            @pl.when(n_own > b)
            def _(b=b):
                i_b = ((n_own - 1 - b) // 2) * 2 + b
                out_desc(i_b, b).wait()

        @pl.when(wid == _NW - 1)
        def _(tab_h=tab_h, out_h=out_h):
            c0 = _NBLK * _CW
            pltpu.make_async_copy(
                tab_h.at[:, pl.ds(c0, _TAILC)], bt, si[0]).start()
            pltpu.make_async_copy(
                tab_h.at[:, pl.ds(c0, _TAILC)], bt, si[0]).wait()
            def body(p, c):
                col = jnp.minimum(2 * p + div8, _TAILC - 1)
                v = plsc.load_gather(bt, [mod8, col])
                b1[0][pl.ds(p * 16, 16)] = v
                return c
            lax.fori_loop(0, (_TAILC * _EMB + 15) // 16, body, 0)
            pltpu.sync_copy(b1[0].at[pl.ds(0, _TAILC * _EMB)],
                            out_h.at[pl.ds(c0 * _EMB, _TAILC * _EMB)])


@functools.partial(
    pl.kernel,
    out_type=jax.ShapeDtypeStruct((48, _B), jnp.float32),
    mesh=plsc.VectorSubcoreMesh(core_axis_name="c", subcore_axis_name="s"),
    compiler_params=pltpu.CompilerParams(
        needs_layout_passes=False, use_tc_tiling_on_sc=False),
    scratch_types=(
        [pltpu.VMEM((_CHI,), jnp.int32) for _ in range(6)]
        + [pltpu.VMEM((_CHI,), jnp.float32) for _ in range(2)]
        + [pltpu.VMEM((_CHI, _EMB), jnp.float32) for _ in range(6)]
        + [pltpu.VMEM((48, _C), jnp.float32)]
        + [pltpu.SemaphoreType.DMA for _ in range(4)]
    ),
)
def _sc_pool(shp_h, clr_h, clu_h, msk_h, ts_h, tc_h, tu_h, out_h,
             i00, i01, i02, i10, i11, i12, m0, m1,
             d00, d01, d02, d10, d11, d12,
             out_v, sem_i0, sem_i1, sem_g0, sem_g1):
    wid = lax.axis_index("s") * _NC + lax.axis_index("c")
    base = wid * _RPW

    idx_hbm = (shp_h, clr_h, clu_h)
    tab_hbm = (ts_h, tc_h, tu_h)
    idx_v = ((i00, i01, i02), (i10, i11, i12))
    msk_v = (m0, m1)
    dat_v = ((d00, d01, d02), (d10, d11, d12))
    sems_i = (sem_i0, sem_i1)
    sems_g = (sem_g0, sem_g1)

    def in_descs(k, b):
        off = (base + k * _C) * _L
        ds = [pltpu.make_async_copy(idx_hbm[t].at[pl.ds(off, _CHI)],
                                    idx_v[b][t], sems_i[b]) for t in range(3)]
        ds.append(pltpu.make_async_copy(msk_h.at[pl.ds(off, _CHI)],
                                        msk_v[b], sems_i[b]))
        return ds

    _SUB = 128
    _NFULL = _CHI // _SUB
    _TAIL = _CHI - _NFULL * _SUB

    def _g_desc(b, t, off, n):
        return pltpu.make_async_copy(
            tab_hbm[t].at[idx_v[b][t].at[pl.ds(off, n)]],
            dat_v[b][t].at[pl.ds(off, n), :], sems_g[b])

    def start_gathers(b):
        def fire(s, c):
            for t in range(3):
                _g_desc(b, t, s * _SUB, _SUB).start()
            return c
        lax.fori_loop(0, _NFULL, fire, 0)
        for t in range(3):
            _g_desc(b, t, _NFULL * _SUB, _TAIL).start()

    def wait_gathers(b):
        def w(s, c):
            for t in range(3):
                _g_desc(b, t, 0, _SUB).wait()
            return c
        lax.fori_loop(0, _NFULL, w, 0)
        for t in range(3):
            _g_desc(b, t, 0, _TAIL).wait()

    def start(descs):
        for d in descs:
            d.start()

    def wait(descs):
        for d in descs:
            d.wait()

    iota = lax.iota(jnp.int32, 16)
    div8 = iota >> 3
    mod8 = iota & 7
    zero = jnp.zeros((16,), jnp.float32)

    def compute(k, b):
        def row_body(r, _):
            roff = r * _L

            def j_body(jc, accs):
                a0, a1, a2 = accs
                base = roff + 8 * jc + div8
                for u in range(4):
                    ridx = base + 2 * u
                    m = plsc.load_gather(msk_v[b], [ridx])
                    d0 = plsc.load_gather(dat_v[b][0], [ridx, mod8])
                    d1 = plsc.load_gather(dat_v[b][1], [ridx, mod8])
                    d2 = plsc.load_gather(dat_v[b][2], [ridx, mod8])
                    a0 = a0 + d0 * m
                    a1 = a1 + d1 * m
                    a2 = a2 + d2 * m
                return (a0, a1, a2)

            a0, a1, a2 = lax.fori_loop(0, _L // 8, j_body, (zero, zero, zero))
            rvec = (iota & 0) + r
            plsc.store_scatter(out_v, [iota, rvec], a0)
            plsc.store_scatter(out_v, [iota + 16, rvec], a1)
            plsc.store_scatter(out_v, [iota + 32, rvec], a2)
            return 0

        lax.fori_loop(0, _C, row_body, 0)
        pltpu.sync_copy(out_v, out_h.at[:, pl.ds(base + k * _C, _C)])

    start(in_descs(0, 0))
    wait(in_descs(0, 0))
    start_gathers(0)
    start(in_descs(1, 1))

    def chunk_pair(k2, carry):
        for par in range(2):
            k = 2 * k2 + par
            b, nb = par, 1 - par

            @pl.when(k + 1 < _NCHUNK)
            def _():
                wait(in_descs(k + 1, nb))
                start_gathers(nb)

            wait_gathers(b)
            compute(k, b)

            @pl.when(k + 2 < _NCHUNK)
            def _():
                start(in_descs(k + 2, b))
        return carry

    lax.fori_loop(0, _NCHUNK // 2, chunk_pair, 0)


_FOLD = np.zeros((48, 24), np.float32)
for _t in range(3):
    for _p in range(2):
        for _d in range(8):
            _FOLD[_t * 16 + _p * 8 + _d, _t * 8 + _d] = 1.0

_BM = 512


def _mlp_body(x_ref, m_ref, w1_ref, b1_ref, w2_ref, b2_ref, o_ref):
    x = x_ref[...]
    xw = jnp.dot(w1_ref[...], x, preferred_element_type=jnp.float32)
    msum = jnp.sum(m_ref[...], axis=0, keepdims=True)
    h = jnp.maximum(xw / msum + b1_ref[...], 0.0)
    o_ref[...] = jnp.dot(w2_ref[...], h, preferred_element_type=jnp.float32) + b2_ref[...]


def kernel(shp, clr, clust, mask, shape_table, color_table, cluster_table,
           W1, b1, W2, b2):
    f0, f1, f2 = _sc_compact(shape_table.T, color_table.T, cluster_table.T)
    x48t = _sc_pool(shp.reshape(-1), clr.reshape(-1), clust.reshape(-1),
                    mask.reshape(-1), f0.reshape(_NV, _EMB),
                    f1.reshape(_NV, _EMB), f2.reshape(_NV, _EMB))
    w1ft = (jnp.asarray(_FOLD) @ W1).T
    ncls = W2.shape[1]
    outt = pl.pallas_call(
        _mlp_body,
        grid=(_B // _BM,),
        in_specs=[
            pl.BlockSpec((48, _BM), lambda i: (0, i)),
            pl.BlockSpec((_L, _BM), lambda i: (0, i)),
            pl.BlockSpec((64, 48), lambda i: (0, 0)),
            pl.BlockSpec((64, 1), lambda i: (0, 0)),
            pl.BlockSpec((ncls, 64), lambda i: (0, 0)),
            pl.BlockSpec((ncls, 1), lambda i: (0, 0)),
        ],
        out_specs=pl.BlockSpec((ncls, _BM), lambda i: (0, i)),
        out_shape=jax.ShapeDtypeStruct((ncls, _B), jnp.float32),
    )(x48t, mask.T, w1ft, b1.reshape(-1, 1), W2.T, b2.reshape(-1, 1))
    return outt.T
